# baseline (speedup 1.0000x reference)
import functools

import jax
import jax.numpy as jnp
from jax import lax
from jax.experimental import pallas as pl
from jax.experimental.pallas import tpu as pltpu
from jax.experimental.pallas import tpu_sc as plsc

_NC = 2
_NS = 16
_NW = _NC * _NS
_L = 16

_H = 64
_KH = _H // _L
_EPS = 1e-12


def _allsum(v):
    for d in (1, 2, 4, 8):
        perm = jnp.arange(_L, dtype=jnp.int32) ^ d
        v = v + v.at[perm].get(mode="promise_in_bounds", unique_indices=True)
    return v


def _rsqrt(x):
    i = lax.bitcast_convert_type(x, jnp.int32)
    i = jnp.int32(0x5F3759DF) - (i >> 1)
    y = lax.bitcast_convert_type(i, jnp.float32)
    xh = 0.5 * x
    for _ in range(2):
        y = y * (1.5 - xh * y * y)
    return y


def _make_sc_kernel(n_rows, seq, chunks_per_worker):
    rows_per_worker = n_rows // _NW
    mesh = plsc.VectorSubcoreMesh(core_axis_name="c", subcore_axis_name="s")

    s0 = min(96, seq)
    s1 = seq - s0
    half = chunks_per_worker // 2

    @functools.partial(
        pl.kernel,
        mesh=mesh,
        out_type=jax.ShapeDtypeStruct((n_rows, _H), jnp.float32),
        compiler_params=pltpu.CompilerParams(use_tc_tiling_on_sc=False),
        scratch_types=[
            pltpu.VMEM((rows_per_worker,), jnp.int32),
            pltpu.VMEM((seq, _H), jnp.float32),
            pltpu.VMEM((seq, _H), jnp.float32),
            pltpu.VMEM((seq, _H), jnp.float32),
            pltpu.VMEM((seq, _H), jnp.float32),
            pltpu.VMEM((seq, _H), jnp.float32),
            pltpu.VMEM((_H,), jnp.float32),
            pltpu.VMEM((_H,), jnp.float32),
            pltpu.SemaphoreType.DMA,
            pltpu.SemaphoreType.DMA,
            pltpu.SemaphoreType.DMA,
            pltpu.SemaphoreType.DMA,
        ],
    )
    def k(ids_hbm, tab_hbm, pos_hbm, g_hbm, b_hbm, out_hbm,
          ids_v, in0, in1, ob0, ob1, pos_v, g_v, b_v,
          gsem0, gsem1, osem0, osem1):
        wid = lax.axis_index("s") * _NC + lax.axis_index("c")
        wbase = wid * rows_per_worker

        pltpu.sync_copy(ids_hbm.at[pl.ds(wbase, rows_per_worker)], ids_v)
        pltpu.sync_copy(pos_hbm, pos_v)
        pltpu.sync_copy(g_hbm, g_v)
        pltpu.sync_copy(b_hbm, b_v)

        g = [g_v[pl.ds(k * _L, _L)] for k in range(_KH)]
        b = [b_v[pl.ds(k * _L, _L)] for k in range(_KH)]

        def gather_descs(c, buf, sem):
            off = c * seq
            d0 = pltpu.make_async_copy(
                tab_hbm.at[ids_v.at[pl.ds(off, s0)]],
                buf.at[pl.ds(0, s0)], sem)
            d1 = pltpu.make_async_copy(
                tab_hbm.at[ids_v.at[pl.ds(off + s0, s1)]],
                buf.at[pl.ds(s0, s1)], sem)
            return d0, d1

        def fire_gather(c, buf, sem):
            d0, d1 = gather_descs(c, buf, sem)
            d0.start()
            d1.start()

        def wait_gather(c, buf, sem):
            d0, d1 = gather_descs(c, buf, sem)
            d0.wait()
            d1.wait()

        def out_desc(c, buf, sem):
            return pltpu.make_async_copy(
                buf, out_hbm.at[pl.ds(wbase + c * seq, seq)], sem)

        unroll = 8

        def compute_chunk(src, dst):
            def _row(r):
                y = [src[r, pl.ds(k * _L, _L)] + pos_v[r, pl.ds(k * _L, _L)]
                     for k in range(_KH)]
                t = (y[0] + y[1]) + (y[2] + y[3])
                u = (y[0] * y[0] + y[1] * y[1]) + (y[2] * y[2] + y[3] * y[3])
                mean = _allsum(t) * (1.0 / _H)
                msq = _allsum(u) * (1.0 / _H)
                var = jnp.maximum(msq - mean * mean, 0.0)
                rstd = _rsqrt(var + _EPS)
                a = [rstd * gk for gk in g]
                for k in range(_KH):
                    dst[r, pl.ds(k * _L, _L)] = (
                        y[k] * a[k] - (mean * a[k] - b[k]))

            def _grp(gi, carry):
                for j in range(unroll):
                    _row(gi * unroll + j)
                return carry

            lax.fori_loop(0, seq // unroll, _grp, 0)

        fire_gather(0, in0, gsem0)

        def loop_body(i, carry):
            c0 = 2 * i
            c1 = c0 + 1

            fire_gather(c1, in1, gsem1)

            wait_gather(c0, in0, gsem0)

            @pl.when(i > 0)
            def _():
                out_desc(c0 - 2, ob0, osem0).wait()

            compute_chunk(in0, ob0)
            out_desc(c0, ob0, osem0).start()

            @pl.when(i < half - 1)
            def _():
                fire_gather(c0 + 2, in0, gsem0)

            wait_gather(c1, in1, gsem1)

            @pl.when(i > 0)
            def _():
                out_desc(c1 - 2, ob1, osem1).wait()

            compute_chunk(in1, ob1)
            out_desc(c1, ob1, osem1).start()

            return carry

        lax.fori_loop(0, half, loop_body, 0)
        out_desc(chunks_per_worker - 2, ob0, osem0).wait()
        out_desc(chunks_per_worker - 1, ob1, osem1).wait()

    return k


def kernel(input_ids, item_table, pos_table, ln_gamma, ln_beta):
    batch, seq = input_ids.shape
    n_rows = batch * seq
    ids = input_ids.reshape(-1).astype(jnp.int32)
    chunks_per_worker = n_rows // (_NW * seq)
    k = _make_sc_kernel(n_rows, seq, chunks_per_worker)
    out = k(ids, item_table, pos_table, ln_gamma, ln_beta)
    return out.reshape(batch, seq, _H)


# speedup vs baseline: 1.0867x; 1.0867x over previous
import functools

import jax
import jax.numpy as jnp
from jax import lax
from jax.experimental import pallas as pl
from jax.experimental.pallas import tpu as pltpu
from jax.experimental.pallas import tpu_sc as plsc

_NC = 2
_NS = 16
_NW = _NC * _NS
_L = 16

_H = 64
_KH = _H // _L
_EPS = 1e-12


def _allsum(v):
    for d in (1, 2, 4, 8):
        perm = jnp.arange(_L, dtype=jnp.int32) ^ d
        v = v + v.at[perm].get(mode="promise_in_bounds", unique_indices=True)
    return v


def _rsqrt(x):
    i = lax.bitcast_convert_type(x, jnp.int32)
    i = jnp.int32(0x5F3759DF) - (i >> 1)
    y = lax.bitcast_convert_type(i, jnp.float32)
    xh = 0.5 * x
    for _ in range(2):
        y = y * (1.5 - xh * y * y)
    return y


def _make_sc_kernel(n_rows, seq, chunks_per_worker):
    rows_per_worker = n_rows // _NW
    mesh = plsc.VectorSubcoreMesh(core_axis_name="c", subcore_axis_name="s")

    s0 = min(96, seq)
    s1 = seq - s0
    half = chunks_per_worker // 2

    @functools.partial(
        pl.kernel,
        mesh=mesh,
        out_type=jax.ShapeDtypeStruct((n_rows, _H), jnp.float32),
        compiler_params=pltpu.CompilerParams(use_tc_tiling_on_sc=False),
        scratch_types=[
            pltpu.VMEM((rows_per_worker,), jnp.int32),
            pltpu.VMEM((seq, _H), jnp.float32),
            pltpu.VMEM((seq, _H), jnp.float32),
            pltpu.VMEM((seq, _H), jnp.float32),
            pltpu.VMEM((seq, _H), jnp.float32),
            pltpu.VMEM((seq, _H), jnp.float32),
            pltpu.VMEM((_H,), jnp.float32),
            pltpu.VMEM((_H,), jnp.float32),
            pltpu.SemaphoreType.DMA,
            pltpu.SemaphoreType.DMA,
            pltpu.SemaphoreType.DMA,
            pltpu.SemaphoreType.DMA,
        ],
    )
    def k(ids_hbm, tab_hbm, pos_hbm, g_hbm, b_hbm, out_hbm,
          ids_v, in0, in1, ob0, ob1, pos_v, g_v, b_v,
          gsem0, gsem1, osem0, osem1):
        wid = lax.axis_index("s") * _NC + lax.axis_index("c")
        wbase = wid * rows_per_worker

        pltpu.sync_copy(ids_hbm.at[pl.ds(wbase, rows_per_worker)], ids_v)
        pltpu.sync_copy(pos_hbm, pos_v)
        pltpu.sync_copy(g_hbm, g_v)
        pltpu.sync_copy(b_hbm, b_v)

        g = [g_v[pl.ds(k * _L, _L)] for k in range(_KH)]
        b = [b_v[pl.ds(k * _L, _L)] for k in range(_KH)]

        def gather_descs(c, buf, sem):
            off = c * seq
            d0 = pltpu.make_async_copy(
                tab_hbm.at[ids_v.at[pl.ds(off, s0)]],
                buf.at[pl.ds(0, s0)], sem)
            d1 = pltpu.make_async_copy(
                tab_hbm.at[ids_v.at[pl.ds(off + s0, s1)]],
                buf.at[pl.ds(s0, s1)], sem)
            return d0, d1

        def fire_gather(c, buf, sem):
            d0, d1 = gather_descs(c, buf, sem)
            d0.start()
            d1.start()

        def wait_gather(c, buf, sem):
            d0, d1 = gather_descs(c, buf, sem)
            d0.wait()
            d1.wait()

        def out_desc(c, buf, sem):
            return pltpu.make_async_copy(
                buf, out_hbm.at[pl.ds(wbase + c * seq, seq)], sem)

        unroll = 4

        def compute_chunk(src, dst):
            def _row(r):
                y = [src[r, pl.ds(k * _L, _L)] + pos_v[r, pl.ds(k * _L, _L)]
                     for k in range(_KH)]
                t = (y[0] + y[1]) + (y[2] + y[3])
                u = (y[0] * y[0] + y[1] * y[1]) + (y[2] * y[2] + y[3] * y[3])
                mean = _allsum(t) * (1.0 / _H)
                msq = _allsum(u) * (1.0 / _H)
                var = jnp.maximum(msq - mean * mean, 0.0)
                rstd = _rsqrt(var + _EPS)
                a = [rstd * gk for gk in g]
                for k in range(_KH):
                    dst[r, pl.ds(k * _L, _L)] = (
                        y[k] * a[k] - (mean * a[k] - b[k]))

            def _grp(gi, carry):
                for j in range(unroll):
                    _row(gi * unroll + j)
                return carry

            lax.fori_loop(0, seq // unroll, _grp, 0)

        fire_gather(0, in0, gsem0)

        def loop_body(i, carry):
            c0 = 2 * i
            c1 = c0 + 1

            fire_gather(c1, in1, gsem1)

            wait_gather(c0, in0, gsem0)

            @pl.when(i > 0)
            def _():
                out_desc(c0 - 2, ob0, osem0).wait()

            compute_chunk(in0, ob0)
            out_desc(c0, ob0, osem0).start()

            @pl.when(i < half - 1)
            def _():
                fire_gather(c0 + 2, in0, gsem0)

            wait_gather(c1, in1, gsem1)

            @pl.when(i > 0)
            def _():
                out_desc(c1 - 2, ob1, osem1).wait()

            compute_chunk(in1, ob1)
            out_desc(c1, ob1, osem1).start()

            return carry

        lax.fori_loop(0, half, loop_body, 0)
        out_desc(chunks_per_worker - 2, ob0, osem0).wait()
        out_desc(chunks_per_worker - 1, ob1, osem1).wait()

    return k


def kernel(input_ids, item_table, pos_table, ln_gamma, ln_beta):
    batch, seq = input_ids.shape
    n_rows = batch * seq
    ids = input_ids.reshape(-1).astype(jnp.int32)
    chunks_per_worker = n_rows // (_NW * seq)
    k = _make_sc_kernel(n_rows, seq, chunks_per_worker)
    out = k(ids, item_table, pos_table, ln_gamma, ln_beta)
    return out.reshape(batch, seq, _H)


# speedup vs baseline: 1.0880x; 1.0012x over previous
import functools

import jax
import jax.numpy as jnp
from jax import lax
from jax.experimental import pallas as pl
from jax.experimental.pallas import tpu as pltpu
from jax.experimental.pallas import tpu_sc as plsc

_NC = 2
_NS = 16
_NW = _NC * _NS
_L = 16

_H = 64
_KH = _H // _L
_EPS = 1e-12


def _allsum(v):
    for d in (1, 2, 4, 8):
        perm = jnp.arange(_L, dtype=jnp.int32) ^ d
        v = v + v.at[perm].get(mode="promise_in_bounds", unique_indices=True)
    return v


def _rsqrt(x):
    i = lax.bitcast_convert_type(x, jnp.int32)
    i = jnp.int32(0x5F3759DF) - (i >> 1)
    y = lax.bitcast_convert_type(i, jnp.float32)
    xh = 0.5 * x
    for _ in range(2):
        y = y * (1.5 - xh * y * y)
    return y


def _make_sc_kernel(n_rows, seq, chunks_per_worker):
    rows_per_worker = n_rows // _NW
    mesh = plsc.VectorSubcoreMesh(core_axis_name="c", subcore_axis_name="s")

    s0 = min(96, seq)
    s1 = seq - s0
    half = chunks_per_worker // 2

    @functools.partial(
        pl.kernel,
        mesh=mesh,
        out_type=jax.ShapeDtypeStruct((n_rows // seq, seq, _H), jnp.float32),
        compiler_params=pltpu.CompilerParams(use_tc_tiling_on_sc=False),
        scratch_types=[
            pltpu.VMEM((rows_per_worker,), jnp.int32),
            pltpu.VMEM((seq, _H), jnp.float32),
            pltpu.VMEM((seq, _H), jnp.float32),
            pltpu.VMEM((seq, _H), jnp.float32),
            pltpu.VMEM((seq, _H), jnp.float32),
            pltpu.VMEM((seq, _H), jnp.float32),
            pltpu.VMEM((_H,), jnp.float32),
            pltpu.VMEM((_H,), jnp.float32),
            pltpu.SemaphoreType.DMA,
            pltpu.SemaphoreType.DMA,
            pltpu.SemaphoreType.DMA,
            pltpu.SemaphoreType.DMA,
        ],
    )
    def k(ids_hbm, tab_hbm, pos_hbm, g_hbm, b_hbm, out_hbm,
          ids_v, in0, in1, ob0, ob1, pos_v, g_v, b_v,
          gsem0, gsem1, osem0, osem1):
        wid = lax.axis_index("s") * _NC + lax.axis_index("c")
        wbase = wid * rows_per_worker

        pltpu.sync_copy(ids_hbm.at[pl.ds(wbase, rows_per_worker)], ids_v)
        pltpu.sync_copy(pos_hbm, pos_v)
        pltpu.sync_copy(g_hbm, g_v)
        pltpu.sync_copy(b_hbm, b_v)

        g = [g_v[pl.ds(k * _L, _L)] for k in range(_KH)]
        b = [b_v[pl.ds(k * _L, _L)] for k in range(_KH)]

        def gather_descs(c, buf, sem):
            off = c * seq
            d0 = pltpu.make_async_copy(
                tab_hbm.at[ids_v.at[pl.ds(off, s0)]],
                buf.at[pl.ds(0, s0)], sem)
            d1 = pltpu.make_async_copy(
                tab_hbm.at[ids_v.at[pl.ds(off + s0, s1)]],
                buf.at[pl.ds(s0, s1)], sem)
            return d0, d1

        def fire_gather(c, buf, sem):
            d0, d1 = gather_descs(c, buf, sem)
            d0.start()
            d1.start()

        def wait_gather(c, buf, sem):
            d0, d1 = gather_descs(c, buf, sem)
            d0.wait()
            d1.wait()

        wseq = wid * chunks_per_worker

        def out_desc(c, buf, sem):
            return pltpu.make_async_copy(buf, out_hbm.at[wseq + c], sem)

        unroll = 4

        def compute_chunk(src, dst):
            def _row(r):
                y = [src[r, pl.ds(k * _L, _L)] + pos_v[r, pl.ds(k * _L, _L)]
                     for k in range(_KH)]
                t = (y[0] + y[1]) + (y[2] + y[3])
                u = (y[0] * y[0] + y[1] * y[1]) + (y[2] * y[2] + y[3] * y[3])
                mean = _allsum(t) * (1.0 / _H)
                msq = _allsum(u) * (1.0 / _H)
                var = jnp.maximum(msq - mean * mean, 0.0)
                rstd = _rsqrt(var + _EPS)
                a = [rstd * gk for gk in g]
                for k in range(_KH):
                    dst[r, pl.ds(k * _L, _L)] = (
                        y[k] * a[k] - (mean * a[k] - b[k]))

            def _grp(gi, carry):
                for j in range(unroll):
                    _row(gi * unroll + j)
                return carry

            lax.fori_loop(0, seq // unroll, _grp, 0)

        fire_gather(0, in0, gsem0)

        def loop_body(i, carry):
            c0 = 2 * i
            c1 = c0 + 1

            fire_gather(c1, in1, gsem1)

            wait_gather(c0, in0, gsem0)

            @pl.when(i > 0)
            def _():
                out_desc(c0 - 2, ob0, osem0).wait()

            compute_chunk(in0, ob0)
            out_desc(c0, ob0, osem0).start()

            @pl.when(i < half - 1)
            def _():
                fire_gather(c0 + 2, in0, gsem0)

            wait_gather(c1, in1, gsem1)

            @pl.when(i > 0)
            def _():
                out_desc(c1 - 2, ob1, osem1).wait()

            compute_chunk(in1, ob1)
            out_desc(c1, ob1, osem1).start()

            return carry

        lax.fori_loop(0, half, loop_body, 0)
        out_desc(chunks_per_worker - 2, ob0, osem0).wait()
        out_desc(chunks_per_worker - 1, ob1, osem1).wait()

    return k


def kernel(input_ids, item_table, pos_table, ln_gamma, ln_beta):
    batch, seq = input_ids.shape
    n_rows = batch * seq
    ids = input_ids.reshape(-1).astype(jnp.int32)
    chunks_per_worker = n_rows // (_NW * seq)
    k = _make_sc_kernel(n_rows, seq, chunks_per_worker)
    return k(ids, item_table, pos_table, ln_gamma, ln_beta)
